# Initial kernel scaffold; baseline (speedup 1.0000x reference)
import jax
import jax.numpy as jnp
from jax.experimental import pallas as pl


def kernel(x, target):
    raise NotImplementedError("write your pallas kernel here")


# speedup vs baseline: 7.7977x; 7.7977x over previous
import math

import jax
import jax.numpy as jnp
from jax.experimental import pallas as pl
from jax.experimental.pallas import tpu as pltpu

N_CLS = 32000
PAD = 0
EPS = 0.1 / (N_CLS - 2)
CONF = 0.9
C0 = (N_CLS - 2) * EPS * math.log(EPS) + CONF * math.log(CONF)

BLK = 1280


def _body(tgt_ref, x_ref, out_ref):
    j = pl.program_id(0)
    x = x_ref[...]
    tgt = tgt_ref[...]
    tmask = tgt != PAD
    col = jax.lax.broadcasted_iota(jnp.int32, (1, BLK), 1) + j * BLK
    w = jnp.where(col == tgt, -CONF, -EPS)
    w = jnp.where(col == PAD, 0.0, w)
    w = jnp.where(tmask, w, 0.0)
    part = jnp.sum(w * x)

    @pl.when(j == 0)
    def _init():
        cnt = jnp.sum(tmask.astype(jnp.float32))
        out_ref[0, 0] = C0 * cnt + part

    @pl.when(j != 0)
    def _acc():
        out_ref[0, 0] += part


def kernel(x, target):
    n, c = x.shape
    out = pl.pallas_call(
        _body,
        grid=(c // BLK,),
        in_specs=[
            pl.BlockSpec((n, 1), lambda j: (0, 0)),
            pl.BlockSpec((n, BLK), lambda j: (0, j)),
        ],
        out_specs=pl.BlockSpec((1, 1), lambda j: (0, 0),
                               memory_space=pltpu.SMEM),
        out_shape=jax.ShapeDtypeStruct((1, 1), jnp.float32),
    )(target.reshape(n, 1), x)
    return out[0, 0]
